# baseline (speedup 1.0000x reference)
import functools

import jax
import jax.numpy as jnp
from jax import lax
from jax.experimental import pallas as pl
from jax.experimental.pallas import tpu as pltpu
from jax.experimental.pallas import tpu_sc as plsc

N_NODES = 10000
N_PAD = 10112
N_EDGES = 320000
D = 128
RPT = N_PAD // 16

NC = 2
NS = 16
NW = NC * NS
E_PER_TILE = N_EDGES // NW
K = 96
NPAIR = 52
TAIL = 16


def _sc_aggregate(x, em, et, zeros48):
    mesh = plsc.VectorSubcoreMesh(core_axis_name="c", subcore_axis_name="s")

    @functools.partial(
        pl.kernel,
        mesh=mesh,
        out_type=[
            jax.ShapeDtypeStruct((NC, N_PAD, D), jnp.float32),
            jax.ShapeDtypeStruct((NC, NS, N_PAD), jnp.float32),
        ],
        compiler_params=pltpu.CompilerParams(use_tc_tiling_on_sc=False,
                                             needs_layout_passes=False),
        scratch_types=[
            pltpu.VMEM((2, 2, K), jnp.int32),
            pltpu.VMEM((2, 2, K), jnp.int32),
            pltpu.VMEM((2, TAIL), jnp.int32),
            pltpu.VMEM((K, D), jnp.float32),
            pltpu.VMEM((K, D), jnp.float32),
            pltpu.VMEM((TAIL, D), jnp.float32),
            pltpu.VMEM((N_PAD,), jnp.float32),
            pltpu.VMEM_SHARED((N_PAD, D), jnp.float32),
            pltpu.SemaphoreType.DMA,
            pltpu.SemaphoreType.DMA,
            pltpu.SemaphoreType.DMA,
        ],
    )
    def k(x_hbm, em_hbm, et_hbm, z_hbm,
          aggr_out, hist_out,
          eidxA, eidxB, tidx, rowsA, rowsB, rowsT, hist_v, aggr_sh,
          semA, semB, semI):
        c = lax.axis_index("c")
        s = lax.axis_index("s")
        wid = s * NC + c

        zeros16 = jnp.zeros((16,), jnp.float32)
        ones16 = jnp.ones((16,), jnp.float32)

        def zh(i, carry):
            hist_v[pl.ds(i * 16, 16)] = zeros16
            return carry

        lax.fori_loop(0, N_PAD // 16, zh, 0)

        pltpu.sync_copy(z_hbm, rowsA)
        for j in range(6):
            pltpu.sync_copy(rowsA, aggr_sh.at[pl.ds(s * RPT + j * K, K)])
        pltpu.sync_copy(rowsA.at[pl.ds(0, 56)],
                        aggr_sh.at[pl.ds(s * RPT + 6 * K, 56)])
        plsc.subcore_barrier()

        def histo(idx_ref, a, b, n):
            for g in range(n // 16):
                dvec = idx_ref[a, b, pl.ds(g * 16, 16)]
                plsc.addupdate_scatter(hist_v, [dvec], ones16)

        def pair_step(p, cur_idx, nxt_idx):
            pf = pltpu.async_copy(em_hbm.at[wid, p + 1], nxt_idx, semI)
            pltpu.make_async_copy(
                x_hbm.at[cur_idx.at[0, 0]], rowsA, semA).wait()
            g2 = pltpu.async_copy(
                x_hbm.at[cur_idx.at[0, 1]], rowsB, semB)
            pltpu.sync_copy(rowsA, aggr_sh.at[cur_idx.at[1, 0]], add=True)
            histo(cur_idx, 1, 0, K)
            pf.wait()
            pltpu.async_copy(
                x_hbm.at[nxt_idx.at[0, 0]], rowsA, semA)
            g2.wait()
            pltpu.sync_copy(rowsB, aggr_sh.at[cur_idx.at[1, 1]], add=True)
            histo(cur_idx, 1, 1, K)

        pltpu.sync_copy(em_hbm.at[wid, 0], eidxA)
        pltpu.async_copy(x_hbm.at[eidxA.at[0, 0]], rowsA, semA)

        def body(j, carry):
            pair_step(2 * j, eidxA, eidxB)
            pair_step(2 * j + 1, eidxB, eidxA)
            return carry

        lax.fori_loop(0, NPAIR // 2, body, 0)

        pltpu.make_async_copy(x_hbm.at[eidxA.at[0, 0]], rowsA, semA).wait()

        pltpu.sync_copy(et_hbm.at[wid], tidx)
        pltpu.async_copy(x_hbm.at[tidx.at[0]], rowsT, semB).wait()
        pltpu.sync_copy(rowsT, aggr_sh.at[tidx.at[1]], add=True)
        dvec_t = tidx[1, pl.ds(0, 16)]
        plsc.addupdate_scatter(hist_v, [dvec_t], ones16)

        pltpu.sync_copy(hist_v, hist_out.at[c, s])

        plsc.subcore_barrier()

        for j in range(6):
            r0 = s * RPT + j * K
            pltpu.sync_copy(aggr_sh.at[pl.ds(r0, K)], rowsA)
            pltpu.sync_copy(rowsA, aggr_out.at[c, pl.ds(r0, K)])
        r0 = s * RPT + 6 * K
        pltpu.sync_copy(aggr_sh.at[pl.ds(r0, 56)], rowsA.at[pl.ds(0, 56)])
        pltpu.sync_copy(rowsA.at[pl.ds(0, 56)], aggr_out.at[c, pl.ds(r0, 56)])

    return k(x, em, et, zeros48)


BLK = 400


def _tc_combine(p, hist, x, W_l, b_l, W_r):
    def body(p_ref, h_ref, x_ref, wl_ref, bl_ref, wr_ref, o_ref):
        ssum = p_ref[0] + p_ref[1]
        deg = jnp.sum(h_ref[...], axis=1)[:, None]
        deg = jnp.maximum(deg, 1.0)
        aggr = ssum / deg
        acc = jnp.dot(aggr, wl_ref[...], preferred_element_type=jnp.float32)
        acc = acc + jnp.dot(x_ref[...], wr_ref[...],
                            preferred_element_type=jnp.float32)
        acc = acc + bl_ref[...]
        o_ref[...] = jnp.maximum(acc, 0.0)

    return pl.pallas_call(
        body,
        grid=(N_NODES // BLK,),
        in_specs=[
            pl.BlockSpec((NC, BLK, D), lambda i: (0, i, 0)),
            pl.BlockSpec((BLK, NC * NS), lambda i: (i, 0)),
            pl.BlockSpec((BLK, D), lambda i: (i, 0)),
            pl.BlockSpec((D, D), lambda i: (0, 0)),
            pl.BlockSpec((1, D), lambda i: (0, 0)),
            pl.BlockSpec((D, D), lambda i: (0, 0)),
        ],
        out_specs=pl.BlockSpec((BLK, D), lambda i: (i, 0)),
        out_shape=jax.ShapeDtypeStruct((N_NODES, D), jnp.float32),
    )(p, hist.reshape(NC * NS, N_PAD).T, x, W_l, b_l.reshape(1, D), W_r)


def kernel(x, edge_index, W_l, b_l, W_r):
    src = edge_index[0].astype(jnp.int32).reshape(NW, E_PER_TILE)
    dst = edge_index[1].astype(jnp.int32).reshape(NW, E_PER_TILE)
    main = NPAIR * 2 * K
    srcm = src[:, :main].reshape(NW, NPAIR, 2, K)
    dstm = dst[:, :main].reshape(NW, NPAIR, 2, K)
    em = jnp.stack([srcm, dstm], axis=2)
    em = jnp.pad(em, ((0, 0), (0, 1), (0, 0), (0, 0), (0, 0)))
    et = jnp.stack([src[:, main:], dst[:, main:]], axis=1)
    zeros48 = jnp.zeros((K, D), jnp.float32)
    p, hist = _sc_aggregate(x, em, et, zeros48)
    return _tc_combine(p, hist, x, W_l, b_l, W_r)


# speedup vs baseline: 1.0689x; 1.0689x over previous
import functools

import jax
import jax.numpy as jnp
from jax import lax
from jax.experimental import pallas as pl
from jax.experimental.pallas import tpu as pltpu
from jax.experimental.pallas import tpu_sc as plsc

N_NODES = 10000
N_PAD = 10112
N_EDGES = 320000
D = 128
RPT = N_PAD // 16

NC = 2
NS = 16
NW = NC * NS
E_PER_TILE = N_EDGES // NW
K = 64
NPAIR = 78
TAIL = 16


def _sc_aggregate(x, em, et, zeros48):
    mesh = plsc.VectorSubcoreMesh(core_axis_name="c", subcore_axis_name="s")

    @functools.partial(
        pl.kernel,
        mesh=mesh,
        out_type=[
            jax.ShapeDtypeStruct((NC, N_PAD, D), jnp.float32),
            jax.ShapeDtypeStruct((NC, NS, N_PAD), jnp.float32),
        ],
        compiler_params=pltpu.CompilerParams(use_tc_tiling_on_sc=False,
                                             needs_layout_passes=False),
        scratch_types=[
            pltpu.VMEM((2, 2, K), jnp.int32),
            pltpu.VMEM((2, 2, K), jnp.int32),
            pltpu.VMEM((2, TAIL), jnp.int32),
            pltpu.VMEM((K, D), jnp.float32),
            pltpu.VMEM((K, D), jnp.float32),
            pltpu.VMEM((TAIL, D), jnp.float32),
            pltpu.VMEM((N_PAD,), jnp.float32),
            pltpu.VMEM_SHARED((N_PAD, D), jnp.float32),
            pltpu.SemaphoreType.DMA,
            pltpu.SemaphoreType.DMA,
            pltpu.SemaphoreType.DMA,
        ],
    )
    def k(x_hbm, em_hbm, et_hbm, z_hbm,
          aggr_out, hist_out,
          eidxA, eidxB, tidx, rowsA, rowsB, rowsT, hist_v, aggr_sh,
          semA, semB, semI):
        c = lax.axis_index("c")
        s = lax.axis_index("s")
        wid = s * NC + c

        zeros16 = jnp.zeros((16,), jnp.float32)
        ones16 = jnp.ones((16,), jnp.float32)

        def zh(i, carry):
            hist_v[pl.ds(i * 16, 16)] = zeros16
            return carry

        lax.fori_loop(0, N_PAD // 16, zh, 0)

        pltpu.sync_copy(z_hbm, rowsA)
        for j in range(9):
            pltpu.sync_copy(rowsA, aggr_sh.at[pl.ds(s * RPT + j * K, K)])
        pltpu.sync_copy(rowsA.at[pl.ds(0, 56)],
                        aggr_sh.at[pl.ds(s * RPT + 9 * K, 56)])
        plsc.subcore_barrier()

        def histo(idx_ref, a, b, n):
            for g in range(n // 16):
                dvec = idx_ref[a, b, pl.ds(g * 16, 16)]
                plsc.addupdate_scatter(hist_v, [dvec], ones16)

        def pair_step(p, cur_idx, nxt_idx):
            pf = pltpu.async_copy(em_hbm.at[wid, p + 1], nxt_idx, semI)
            pltpu.make_async_copy(
                x_hbm.at[cur_idx.at[0, 0]], rowsA, semA).wait()
            g2 = pltpu.async_copy(
                x_hbm.at[cur_idx.at[0, 1]], rowsB, semB)
            pltpu.sync_copy(rowsA, aggr_sh.at[cur_idx.at[1, 0]], add=True)
            histo(cur_idx, 1, 0, K)
            pf.wait()
            pltpu.async_copy(
                x_hbm.at[nxt_idx.at[0, 0]], rowsA, semA)
            g2.wait()
            pltpu.sync_copy(rowsB, aggr_sh.at[cur_idx.at[1, 1]], add=True)
            histo(cur_idx, 1, 1, K)

        pltpu.sync_copy(em_hbm.at[wid, 0], eidxA)
        pltpu.async_copy(x_hbm.at[eidxA.at[0, 0]], rowsA, semA)

        def body(j, carry):
            pair_step(2 * j, eidxA, eidxB)
            pair_step(2 * j + 1, eidxB, eidxA)
            return carry

        lax.fori_loop(0, NPAIR // 2, body, 0)

        pltpu.make_async_copy(x_hbm.at[eidxA.at[0, 0]], rowsA, semA).wait()

        pltpu.sync_copy(et_hbm.at[wid], tidx)
        pltpu.async_copy(x_hbm.at[tidx.at[0]], rowsT, semB).wait()
        pltpu.sync_copy(rowsT, aggr_sh.at[tidx.at[1]], add=True)
        dvec_t = tidx[1, pl.ds(0, 16)]
        plsc.addupdate_scatter(hist_v, [dvec_t], ones16)

        pltpu.sync_copy(hist_v, hist_out.at[c, s])

        plsc.subcore_barrier()

        for j in range(9):
            r0 = s * RPT + j * K
            pltpu.sync_copy(aggr_sh.at[pl.ds(r0, K)], rowsA)
            pltpu.sync_copy(rowsA, aggr_out.at[c, pl.ds(r0, K)])
        r0 = s * RPT + 9 * K
        pltpu.sync_copy(aggr_sh.at[pl.ds(r0, 56)], rowsA.at[pl.ds(0, 56)])
        pltpu.sync_copy(rowsA.at[pl.ds(0, 56)], aggr_out.at[c, pl.ds(r0, 56)])

    return k(x, em, et, zeros48)


BLK = 400


def _tc_combine(p, hist, x, W_l, b_l, W_r):
    def body(p_ref, h_ref, x_ref, wl_ref, bl_ref, wr_ref, o_ref):
        ssum = p_ref[0] + p_ref[1]
        deg = jnp.sum(h_ref[...], axis=1)[:, None]
        deg = jnp.maximum(deg, 1.0)
        aggr = ssum / deg
        acc = jnp.dot(aggr, wl_ref[...], preferred_element_type=jnp.float32)
        acc = acc + jnp.dot(x_ref[...], wr_ref[...],
                            preferred_element_type=jnp.float32)
        acc = acc + bl_ref[...]
        o_ref[...] = jnp.maximum(acc, 0.0)

    return pl.pallas_call(
        body,
        grid=(N_NODES // BLK,),
        in_specs=[
            pl.BlockSpec((NC, BLK, D), lambda i: (0, i, 0)),
            pl.BlockSpec((BLK, NC * NS), lambda i: (i, 0)),
            pl.BlockSpec((BLK, D), lambda i: (i, 0)),
            pl.BlockSpec((D, D), lambda i: (0, 0)),
            pl.BlockSpec((1, D), lambda i: (0, 0)),
            pl.BlockSpec((D, D), lambda i: (0, 0)),
        ],
        out_specs=pl.BlockSpec((BLK, D), lambda i: (i, 0)),
        out_shape=jax.ShapeDtypeStruct((N_NODES, D), jnp.float32),
    )(p, hist.reshape(NC * NS, N_PAD).T, x, W_l, b_l.reshape(1, D), W_r)


def kernel(x, edge_index, W_l, b_l, W_r):
    src = edge_index[0].astype(jnp.int32).reshape(NW, E_PER_TILE)
    dst = edge_index[1].astype(jnp.int32).reshape(NW, E_PER_TILE)
    main = NPAIR * 2 * K
    srcm = src[:, :main].reshape(NW, NPAIR, 2, K)
    dstm = dst[:, :main].reshape(NW, NPAIR, 2, K)
    em = jnp.stack([srcm, dstm], axis=2)
    em = jnp.pad(em, ((0, 0), (0, 1), (0, 0), (0, 0), (0, 0)))
    et = jnp.stack([src[:, main:], dst[:, main:]], axis=1)
    zeros48 = jnp.zeros((K, D), jnp.float32)
    p, hist = _sc_aggregate(x, em, et, zeros48)
    return _tc_combine(p, hist, x, W_l, b_l, W_r)
